# baseline (speedup 1.0000x reference)
import functools

import jax
import jax.numpy as jnp
from jax import lax
from jax.experimental import pallas as pl
from jax.experimental.pallas import tpu as pltpu
from jax.experimental.pallas import tpu_sc as plsc

TOKEN_UNIVERSE = 120000
TABLE_SIZE = TOKEN_UNIVERSE + 64
NUM_OOV = 1
NUM_WORKERS = 32
LANES = 16
COL_W = 512
CHUNK_ROWS = 4
VCHUNK_ROWS = 8
VOCAB_PAD = 102400


def _unrolled(n_total, unroll, body):
    assert n_total % unroll == 0

    def outer(o, _):
        for u in range(unroll):
            body(o * jnp.int32(unroll) + jnp.int32(u), u)
        return _

    lax.fori_loop(jnp.int32(0), jnp.int32(n_total // unroll), outer, None)


def _sc_lookup(tok_t, voc2d):
    t_rows, t_cols = tok_t.shape
    v_rows = voc2d.shape[0]
    n_chunks = t_rows // CHUNK_ROWS
    v_chunks = v_rows // VCHUNK_ROWS
    assert t_cols == COL_W * NUM_WORKERS
    assert n_chunks * CHUNK_ROWS == t_rows
    assert v_chunks * VCHUNK_ROWS == v_rows
    assert n_chunks % 2 == 0 and v_chunks % 2 == 0

    mesh = plsc.VectorSubcoreMesh(
        core_axis_name="c", subcore_axis_name="s", num_cores=2, num_subcores=16
    )

    @functools.partial(
        pl.kernel,
        out_type=jax.ShapeDtypeStruct(tok_t.shape, jnp.int32),
        mesh=mesh,
        compiler_params=pltpu.CompilerParams(
            needs_layout_passes=False, disable_bounds_checks=True),
        scratch_types=[
            pltpu.VMEM((TABLE_SIZE,), jnp.int32),
            pltpu.VMEM((VCHUNK_ROWS, 128), jnp.uint32),
            pltpu.VMEM((VCHUNK_ROWS, 128), jnp.uint32),
            pltpu.VMEM((CHUNK_ROWS, COL_W), jnp.int32),
            pltpu.VMEM((CHUNK_ROWS, COL_W), jnp.int32),
            pltpu.SemaphoreType.DMA,
            pltpu.SemaphoreType.DMA,
            pltpu.SemaphoreType.DMA,
            pltpu.SemaphoreType.DMA,
            pltpu.SemaphoreType.DMA,
            pltpu.SemaphoreType.DMA,
        ],
    )
    def k(tok_hbm, voc_hbm, out_hbm, table_v, vb0, vb1, ib0, ib1,
          sv0, sv1, si0, si1, so0, so1):
        lane = lax.iota(jnp.int32, LANES)
        vbs, ibs = (vb0, vb1), (ib0, ib1)
        svs, sis, sos = (sv0, sv1), (si0, si1), (so0, so1)

        wid = lax.axis_index("s") * 2 + lax.axis_index("c")
        col0 = wid * jnp.int32(COL_W)

        def vslice(c):
            return voc_hbm.at[pl.ds(c * VCHUNK_ROWS, VCHUNK_ROWS), :]

        def tslice(c):
            return tok_hbm.at[pl.ds(c * CHUNK_ROWS, CHUNK_ROWS),
                              pl.ds(col0, COL_W)]

        def oslice(c):
            return out_hbm.at[pl.ds(c * CHUNK_ROWS, CHUNK_ROWS),
                              pl.ds(col0, COL_W)]

        pltpu.async_copy(vslice(jnp.int32(0)), vb0, sv0)
        pltpu.async_copy(tslice(jnp.int32(0)), ib0, si0)

        zeros = jnp.zeros((LANES,), jnp.int32)

        def zero_body(i, _u):
            table_v[pl.ds(i * LANES, LANES)] = zeros

        _unrolled(TABLE_SIZE // LANES, 8, zero_body)

        def build_pair(c2, _):
            for b in (0, 1):
                c = c2 * 2 + jnp.int32(b)
                if b == 0:
                    pltpu.async_copy(vslice(c + 1), vbs[1], svs[1])
                else:
                    @pl.when(c2 < v_chunks // 2 - 1)
                    def _():
                        pltpu.async_copy(vslice(c + 1), vbs[0], svs[0])
                pltpu.make_async_copy(vslice(c), vbs[b], svs[b]).wait()
                vbase = c * jnp.int32(VCHUNK_ROWS * 128) + NUM_OOV + lane
                for r in range(VCHUNK_ROWS):
                    for u in range(8):
                        ids = plsc.bitcast(
                            vbs[b][r, pl.ds(u * LANES, LANES)], jnp.int32)
                        plsc.store_scatter(
                            table_v, [ids],
                            vbase + jnp.int32((r * 8 + u) * LANES))
            return _

        lax.fori_loop(jnp.int32(0), jnp.int32(v_chunks // 2), build_pair, None)

        def lookup_pair(c2, _):
            for b in (0, 1):
                c = c2 * 2 + jnp.int32(b)
                if b == 0:
                    @pl.when(c2 >= 1)
                    def _():
                        pltpu.make_async_copy(
                            ibs[1], oslice(c - 1), sos[1]).wait()
                    pltpu.async_copy(tslice(c + 1), ibs[1], sis[1])
                else:
                    @pl.when(c2 < n_chunks // 2 - 1)
                    def _():
                        pltpu.make_async_copy(
                            ibs[0], oslice(c - 1), sos[0]).wait()
                        pltpu.async_copy(tslice(c + 1), ibs[0], sis[0])
                pltpu.make_async_copy(tslice(c), ibs[b], sis[b]).wait()

                for r in range(CHUNK_ROWS):
                    for u in range(COL_W // LANES):
                        t = ibs[b][r, pl.ds(u * LANES, LANES)]
                        ibs[b][r, pl.ds(u * LANES, LANES)] = plsc.load_gather(
                            table_v, [t])
                pltpu.async_copy(ibs[b], oslice(c), sos[b])
            return _

        lax.fori_loop(jnp.int32(0), jnp.int32(n_chunks // 2), lookup_pair, None)

        pltpu.make_async_copy(ib0, oslice(jnp.int32(n_chunks - 2)), so0).wait()
        pltpu.make_async_copy(ib1, oslice(jnp.int32(n_chunks - 1)), so1).wait()

    return k(tok_t, voc2d)


def kernel(tokens, vocab):
    tok_t = lax.bitcast_convert_type(tokens.T.astype(jnp.uint32), jnp.int32)
    voc32 = vocab.astype(jnp.uint32)
    pad = jnp.full((VOCAB_PAD - voc32.shape[0],), TOKEN_UNIVERSE, jnp.uint32)
    voc2d = jnp.concatenate([voc32, pad]).reshape(VOCAB_PAD // 128, 128)
    out_t = _sc_lookup(tok_t, voc2d)
    return lax.bitcast_convert_type(out_t.T, jnp.uint32).astype(tokens.dtype)


# speedup vs baseline: 1.0488x; 1.0488x over previous
import functools

import jax
import jax.numpy as jnp
from jax import lax
from jax.experimental import pallas as pl
from jax.experimental.pallas import tpu as pltpu
from jax.experimental.pallas import tpu_sc as plsc

TOKEN_UNIVERSE = 120000
TABLE_SIZE = TOKEN_UNIVERSE + 64
NUM_OOV = 1
NUM_WORKERS = 32
LANES = 16
COL_W = 512
CHUNK_ROWS = 4
VCHUNK_ROWS = 8
VOCAB_PAD = 102400


def _unrolled(n_total, unroll, body):
    assert n_total % unroll == 0

    def outer(o, _):
        for u in range(unroll):
            body(o * jnp.int32(unroll) + jnp.int32(u), u)
        return _

    lax.fori_loop(jnp.int32(0), jnp.int32(n_total // unroll), outer, None)


def _sc_lookup(tok_t, voc2d):
    t_rows, t_cols = tok_t.shape
    v_rows = voc2d.shape[0]
    n_chunks = t_rows // CHUNK_ROWS
    v_chunks = v_rows // VCHUNK_ROWS
    assert t_cols == COL_W * NUM_WORKERS
    assert n_chunks * CHUNK_ROWS == t_rows
    assert v_chunks * VCHUNK_ROWS == v_rows
    assert n_chunks % 2 == 0 and v_chunks % 2 == 0

    mesh = plsc.VectorSubcoreMesh(
        core_axis_name="c", subcore_axis_name="s", num_cores=2, num_subcores=16
    )

    @functools.partial(
        pl.kernel,
        out_type=jax.ShapeDtypeStruct(tok_t.shape, jnp.uint32),
        mesh=mesh,
        compiler_params=pltpu.CompilerParams(
            needs_layout_passes=False, disable_bounds_checks=True),
        scratch_types=[
            pltpu.VMEM((TABLE_SIZE,), jnp.int32),
            pltpu.VMEM((VCHUNK_ROWS, 128), jnp.uint32),
            pltpu.VMEM((VCHUNK_ROWS, 128), jnp.uint32),
            pltpu.VMEM((CHUNK_ROWS, COL_W), jnp.uint32),
            pltpu.VMEM((CHUNK_ROWS, COL_W), jnp.uint32),
            pltpu.SemaphoreType.DMA,
            pltpu.SemaphoreType.DMA,
            pltpu.SemaphoreType.DMA,
            pltpu.SemaphoreType.DMA,
            pltpu.SemaphoreType.DMA,
            pltpu.SemaphoreType.DMA,
        ],
    )
    def k(tok_hbm, voc_hbm, out_hbm, table_v, vb0, vb1, ib0, ib1,
          sv0, sv1, si0, si1, so0, so1):
        lane = lax.iota(jnp.int32, LANES)
        vbs, ibs = (vb0, vb1), (ib0, ib1)
        svs, sis, sos = (sv0, sv1), (si0, si1), (so0, so1)

        wid = lax.axis_index("s") * 2 + lax.axis_index("c")
        col0 = wid * jnp.int32(COL_W)

        def vslice(c):
            return voc_hbm.at[pl.ds(c * VCHUNK_ROWS, VCHUNK_ROWS), :]

        def tslice(c):
            return tok_hbm.at[pl.ds(c * CHUNK_ROWS, CHUNK_ROWS),
                              pl.ds(col0, COL_W)]

        def oslice(c):
            return out_hbm.at[pl.ds(c * CHUNK_ROWS, CHUNK_ROWS),
                              pl.ds(col0, COL_W)]

        pltpu.async_copy(vslice(jnp.int32(0)), vb0, sv0)
        pltpu.async_copy(tslice(jnp.int32(0)), ib0, si0)

        zeros = jnp.zeros((LANES,), jnp.int32)

        def zero_body(i, _u):
            table_v[pl.ds(i * LANES, LANES)] = zeros

        _unrolled(TABLE_SIZE // LANES, 8, zero_body)

        def build_pair(c2, _):
            for b in (0, 1):
                c = c2 * 2 + jnp.int32(b)
                if b == 0:
                    pltpu.async_copy(vslice(c + 1), vbs[1], svs[1])
                else:
                    @pl.when(c2 < v_chunks // 2 - 1)
                    def _():
                        pltpu.async_copy(vslice(c + 1), vbs[0], svs[0])
                pltpu.make_async_copy(vslice(c), vbs[b], svs[b]).wait()
                vbase = c * jnp.int32(VCHUNK_ROWS * 128) + NUM_OOV + lane
                for r in range(VCHUNK_ROWS):
                    for u in range(8):
                        ids = plsc.bitcast(
                            vbs[b][r, pl.ds(u * LANES, LANES)], jnp.int32)
                        plsc.store_scatter(
                            table_v, [ids],
                            vbase + jnp.int32((r * 8 + u) * LANES))
            return _

        lax.fori_loop(jnp.int32(0), jnp.int32(v_chunks // 2), build_pair, None)

        def lookup_pair(c2, _):
            for b in (0, 1):
                c = c2 * 2 + jnp.int32(b)
                if b == 0:
                    @pl.when(c2 >= 1)
                    def _():
                        pltpu.make_async_copy(
                            ibs[1], oslice(c - 1), sos[1]).wait()
                    pltpu.async_copy(tslice(c + 1), ibs[1], sis[1])
                else:
                    @pl.when(c2 < n_chunks // 2 - 1)
                    def _():
                        pltpu.make_async_copy(
                            ibs[0], oslice(c - 1), sos[0]).wait()
                        pltpu.async_copy(tslice(c + 1), ibs[0], sis[0])
                pltpu.make_async_copy(tslice(c), ibs[b], sis[b]).wait()

                for r in range(CHUNK_ROWS):
                    for u in range(COL_W // LANES):
                        t = plsc.bitcast(
                            ibs[b][r, pl.ds(u * LANES, LANES)], jnp.int32)
                        ibs[b][r, pl.ds(u * LANES, LANES)] = plsc.bitcast(
                            plsc.load_gather(table_v, [t]), jnp.uint32)
                pltpu.async_copy(ibs[b], oslice(c), sos[b])
            return _

        lax.fori_loop(jnp.int32(0), jnp.int32(n_chunks // 2), lookup_pair, None)

        pltpu.make_async_copy(ib0, oslice(jnp.int32(n_chunks - 2)), so0).wait()
        pltpu.make_async_copy(ib1, oslice(jnp.int32(n_chunks - 1)), so1).wait()

    return k(tok_t, voc2d)


def kernel(tokens, vocab):
    tok_t = tokens.T.astype(jnp.uint32)
    voc32 = vocab.astype(jnp.uint32)
    pad = jnp.full((VOCAB_PAD - voc32.shape[0],), TOKEN_UNIVERSE, jnp.uint32)
    voc2d = jnp.concatenate([voc32, pad]).reshape(VOCAB_PAD // 128, 128)
    out_t = _sc_lookup(tok_t, voc2d)
    return out_t.T.astype(tokens.dtype)


# speedup vs baseline: 1.1685x; 1.1141x over previous
import functools

import jax
import jax.numpy as jnp
from jax import lax
from jax.experimental import pallas as pl
from jax.experimental.pallas import tpu as pltpu
from jax.experimental.pallas import tpu_sc as plsc

TOKEN_UNIVERSE = 120000
NUM_WORKERS = 32
LANES = 16
SHARD_ROWS = 32
TABLE_HBM_ROWS = SHARD_ROWS * NUM_WORKERS
TABLE_ROWS = 944
NUM_OOV = 1
COL_W = 512
CHUNK_ROWS = 4
VCHUNK_ROWS = 8
VOCAB_PAD = 102400

_MESH = dict(core_axis_name="c", subcore_axis_name="s",
             num_cores=2, num_subcores=16)


def _build_table(voc2d):
    v_rows = voc2d.shape[0]
    v_chunks = v_rows // VCHUNK_ROWS
    assert v_chunks * VCHUNK_ROWS == v_rows and v_chunks % 2 == 0

    @functools.partial(
        pl.kernel,
        out_type=jax.ShapeDtypeStruct((TABLE_HBM_ROWS, 128), jnp.int32),
        mesh=plsc.VectorSubcoreMesh(**_MESH),
        compiler_params=pltpu.CompilerParams(
            needs_layout_passes=False, disable_bounds_checks=True),
        scratch_types=[
            pltpu.VMEM((SHARD_ROWS, 128), jnp.int32),
            pltpu.VMEM((VCHUNK_ROWS, 128), jnp.uint32),
            pltpu.VMEM((VCHUNK_ROWS, 128), jnp.uint32),
            pltpu.SemaphoreType.DMA,
            pltpu.SemaphoreType.DMA,
            pltpu.SemaphoreType.DMA,
        ],
    )
    def k(voc_hbm, tab_hbm, shard_v, vb0, vb1, sv0, sv1, st):
        lane = lax.iota(jnp.int32, LANES)
        vbs, svs = (vb0, vb1), (sv0, sv1)

        wid = lax.axis_index("s") * 2 + lax.axis_index("c")
        base = wid * jnp.int32(SHARD_ROWS * 128)

        def vslice(c):
            return voc_hbm.at[pl.ds(c * VCHUNK_ROWS, VCHUNK_ROWS), :]

        pltpu.async_copy(vslice(jnp.int32(0)), vb0, sv0)

        zeros = jnp.zeros((LANES,), jnp.int32)
        for r in range(SHARD_ROWS):
            for u in range(8):
                shard_v[r, pl.ds(u * LANES, LANES)] = zeros

        def build_pair(c2, _):
            for b in (0, 1):
                c = c2 * 2 + jnp.int32(b)
                if b == 0:
                    pltpu.async_copy(vslice(c + 1), vbs[1], svs[1])
                else:
                    @pl.when(c2 < v_chunks // 2 - 1)
                    def _():
                        pltpu.async_copy(vslice(c + 1), vbs[0], svs[0])
                pltpu.make_async_copy(vslice(c), vbs[b], svs[b]).wait()
                vbase = c * jnp.int32(VCHUNK_ROWS * 128) + NUM_OOV + lane
                for r in range(VCHUNK_ROWS):
                    for u in range(8):
                        ids = plsc.bitcast(
                            vbs[b][r, pl.ds(u * LANES, LANES)], jnp.int32)
                        off = ids - base
                        m = (off >= 0) & (off < SHARD_ROWS * 128)
                        offc = jnp.where(m, off, 0)
                        plsc.store_scatter(
                            shard_v, [offc >> 7, offc & 127],
                            vbase + jnp.int32((r * 8 + u) * LANES), mask=m)
            return _

        lax.fori_loop(jnp.int32(0), jnp.int32(v_chunks // 2), build_pair, None)

        dst = tab_hbm.at[pl.ds(wid * jnp.int32(SHARD_ROWS), SHARD_ROWS), :]
        pltpu.async_copy(shard_v, dst, st)
        pltpu.make_async_copy(shard_v, dst, st).wait()

    return k(voc2d)


def _lookup(tok_t, tab):
    t_rows, t_cols = tok_t.shape
    n_chunks = t_rows // CHUNK_ROWS
    assert t_cols == COL_W * NUM_WORKERS
    assert n_chunks * CHUNK_ROWS == t_rows and n_chunks % 2 == 0

    @functools.partial(
        pl.kernel,
        out_type=jax.ShapeDtypeStruct(tok_t.shape, jnp.uint32),
        mesh=plsc.VectorSubcoreMesh(**_MESH),
        compiler_params=pltpu.CompilerParams(
            needs_layout_passes=False, disable_bounds_checks=True),
        scratch_types=[
            pltpu.VMEM((TABLE_ROWS, 128), jnp.int32),
            pltpu.VMEM((CHUNK_ROWS, COL_W), jnp.uint32),
            pltpu.VMEM((CHUNK_ROWS, COL_W), jnp.uint32),
            pltpu.VMEM((CHUNK_ROWS, COL_W), jnp.uint32),
            pltpu.VMEM((CHUNK_ROWS, COL_W), jnp.uint32),
            pltpu.SemaphoreType.DMA,
            pltpu.SemaphoreType.DMA,
            pltpu.SemaphoreType.DMA,
            pltpu.SemaphoreType.DMA,
            pltpu.SemaphoreType.DMA,
        ],
    )
    def k(tok_hbm, tab_hbm, out_hbm, table_v, ib0, ib1, ob0, ob1,
          st, si0, si1, so0, so1):
        ibs, obs = (ib0, ib1), (ob0, ob1)
        sis, sos = (si0, si1), (so0, so1)

        wid = lax.axis_index("s") * 2 + lax.axis_index("c")
        col0 = wid * jnp.int32(COL_W)

        def tslice(c):
            return tok_hbm.at[pl.ds(c * CHUNK_ROWS, CHUNK_ROWS),
                              pl.ds(col0, COL_W)]

        def oslice(c):
            return out_hbm.at[pl.ds(c * CHUNK_ROWS, CHUNK_ROWS),
                              pl.ds(col0, COL_W)]

        pltpu.async_copy(tslice(jnp.int32(0)), ib0, si0)
        pltpu.async_copy(tslice(jnp.int32(1)), ib1, si1)
        tsrc = tab_hbm.at[pl.ds(0, TABLE_ROWS), :]
        pltpu.async_copy(tsrc, table_v, st)
        pltpu.make_async_copy(tsrc, table_v, st).wait()

        def lookup_pair(c2, _):
            for b in (0, 1):
                c = c2 * 2 + jnp.int32(b)

                @pl.when(c2 >= 1)
                def _():
                    pltpu.make_async_copy(obs[b], oslice(c), sos[b]).wait()
                pltpu.make_async_copy(tslice(c), ibs[b], sis[b]).wait()

                for r in range(CHUNK_ROWS):
                    for u in range(COL_W // LANES):
                        t = plsc.bitcast(
                            ibs[b][r, pl.ds(u * LANES, LANES)], jnp.int32)
                        obs[b][r, pl.ds(u * LANES, LANES)] = plsc.bitcast(
                            plsc.load_gather(table_v, [t >> 7, t & 127]),
                            jnp.uint32)
                pltpu.async_copy(obs[b], oslice(c), sos[b])

                @pl.when(c2 < n_chunks // 2 - 1)
                def _():
                    pltpu.async_copy(tslice(c + 2), ibs[b], sis[b])
            return _

        lax.fori_loop(jnp.int32(0), jnp.int32(n_chunks // 2), lookup_pair, None)

        pltpu.make_async_copy(ob0, oslice(jnp.int32(n_chunks - 2)), so0).wait()
        pltpu.make_async_copy(ob1, oslice(jnp.int32(n_chunks - 1)), so1).wait()

    return k(tok_t, tab)


def kernel(tokens, vocab):
    tok_t = tokens.T.astype(jnp.uint32)
    voc32 = vocab.astype(jnp.uint32)
    pad = jnp.full((VOCAB_PAD - voc32.shape[0],), TOKEN_UNIVERSE, jnp.uint32)
    voc2d = jnp.concatenate([voc32, pad]).reshape(VOCAB_PAD // 128, 128)
    tab = _build_table(voc2d)
    out_t = _lookup(tok_t, tab)
    return out_t.T.astype(tokens.dtype)
